# Initial kernel scaffold; baseline (speedup 1.0000x reference)
import jax
import jax.numpy as jnp
from jax.experimental import pallas as pl


def kernel(x, edge_index, edge_attr, W1, att_src1, att_dst1, W_e1, att_e1, b1, W2, att_src2, att_dst2, W_e2, att_e2, b2):
    raise NotImplementedError("write your pallas kernel here")


# speedup vs baseline: 17.9701x; 17.9701x over previous
import functools

import jax
import jax.numpy as jnp
from jax import lax
from jax.experimental import pallas as pl
from jax.experimental.pallas import tpu as pltpu
from jax.experimental.pallas import tpu_sc as plsc

_L = 16
_NSUB = 16
_NCORE = 2
_NW = _NSUB * _NCORE
_CHUNK = 128
_BLK = 2000


def _dense1_body(x_ref, w_ref, asr_ref, adr_ref, we_ref, ate_ref,
                 h_ref, s_ref, d_ref, ce_ref):
    h = jnp.dot(x_ref[...], w_ref[...], preferred_element_type=jnp.float32)
    h_ref[...] = h
    s_ref[...] = jnp.dot(h, asr_ref[...], preferred_element_type=jnp.float32)
    d_ref[...] = jnp.dot(h, adr_ref[...], preferred_element_type=jnp.float32)
    ce_ref[...] = jnp.full((1, 128), jnp.sum(we_ref[...] * ate_ref[...]),
                           dtype=jnp.float32)


def _mid_body(num_ref, den_ref, b_ref, w_ref, asr_ref, adr_ref, we_ref, ate_ref,
              h_ref, s_ref, d_ref, ce_ref):
    nm = num_ref[0] + num_ref[1]
    dn = jnp.sum(den_ref[...], axis=1, keepdims=True)
    h1 = nm / (dn + 1e-16) + b_ref[...]
    h1 = jnp.maximum(h1, 0.0)
    h = jnp.dot(h1, w_ref[...], preferred_element_type=jnp.float32)
    h_ref[...] = h
    s_ref[...] = jnp.dot(h, asr_ref[...], preferred_element_type=jnp.float32)
    d_ref[...] = jnp.dot(h, adr_ref[...], preferred_element_type=jnp.float32)
    ce_ref[...] = jnp.full((1, 128), jnp.sum(we_ref[...] * ate_ref[...]),
                           dtype=jnp.float32)


def _final_body(num_ref, den_ref, b_ref, out_ref):
    nm = num_ref[0] + num_ref[1]
    dn = jnp.sum(den_ref[...], axis=1, keepdims=True)
    out_ref[...] = nm / (dn + 1e-16) + b_ref[...]


def _run_dense1(x, W, att_s, att_d, We, att_e, n):
    grid = n // _BLK
    full = lambda i: (0, 0)
    row = lambda i: (i, 0)
    return pl.pallas_call(
        _dense1_body,
        grid=(grid,),
        in_specs=[
            pl.BlockSpec((_BLK, 128), row),
            pl.BlockSpec((128, 128), full),
            pl.BlockSpec((128, 1), full),
            pl.BlockSpec((128, 1), full),
            pl.BlockSpec((1, 128), full),
            pl.BlockSpec((1, 128), full),
        ],
        out_specs=[
            pl.BlockSpec((_BLK, 128), row),
            pl.BlockSpec((_BLK, 1), row),
            pl.BlockSpec((_BLK, 1), row),
            pl.BlockSpec((1, 128), full),
        ],
        out_shape=[
            jax.ShapeDtypeStruct((n, 128), jnp.float32),
            jax.ShapeDtypeStruct((n, 1), jnp.float32),
            jax.ShapeDtypeStruct((n, 1), jnp.float32),
            jax.ShapeDtypeStruct((1, 128), jnp.float32),
        ],
    )(x, W, att_s, att_d, We, att_e)


def _run_mid(num, den_t, b, W, att_s, att_d, We, att_e, n):
    grid = n // _BLK
    full = lambda i: (0, 0)
    row = lambda i: (i, 0)
    return pl.pallas_call(
        _mid_body,
        grid=(grid,),
        in_specs=[
            pl.BlockSpec((2, _BLK, 128), lambda i: (0, i, 0)),
            pl.BlockSpec((_BLK, 32), row),
            pl.BlockSpec((1, 128), full),
            pl.BlockSpec((128, 128), full),
            pl.BlockSpec((128, 1), full),
            pl.BlockSpec((128, 1), full),
            pl.BlockSpec((1, 128), full),
            pl.BlockSpec((1, 128), full),
        ],
        out_specs=[
            pl.BlockSpec((_BLK, 128), row),
            pl.BlockSpec((_BLK, 1), row),
            pl.BlockSpec((_BLK, 1), row),
            pl.BlockSpec((1, 128), full),
        ],
        out_shape=[
            jax.ShapeDtypeStruct((n, 128), jnp.float32),
            jax.ShapeDtypeStruct((n, 1), jnp.float32),
            jax.ShapeDtypeStruct((n, 1), jnp.float32),
            jax.ShapeDtypeStruct((1, 128), jnp.float32),
        ],
    )(num, den_t, b, W, att_s, att_d, We, att_e)


def _run_final(num, den_t, b, n):
    grid = n // _BLK
    full = lambda i: (0, 0)
    row = lambda i: (i, 0)
    return pl.pallas_call(
        _final_body,
        grid=(grid,),
        in_specs=[
            pl.BlockSpec((2, _BLK, 128), lambda i: (0, i, 0)),
            pl.BlockSpec((_BLK, 32), row),
            pl.BlockSpec((1, 128), full),
        ],
        out_specs=pl.BlockSpec((_BLK, 128), row),
        out_shape=jax.ShapeDtypeStruct((n, 128), jnp.float32),
    )(num, den_t, b)


def _make_edge_kernel(n, e, ept):
    nchunk = ept // _CHUNK
    npad = -(-n // (_NSUB * _CHUNK)) * (_NSUB * _CHUNK)
    rows_per_tile = npad // _NSUB
    nfull = rows_per_tile // _CHUNK

    mesh = plsc.VectorSubcoreMesh(core_axis_name="c", subcore_axis_name="s")

    @functools.partial(
        pl.kernel,
        out_type=[
            jax.ShapeDtypeStruct((_NCORE * npad, 128), jnp.float32),
            jax.ShapeDtypeStruct((_NW * n,), jnp.float32),
        ],
        mesh=mesh,
        scratch_types=[
            pltpu.VMEM((n,), jnp.float32),
            pltpu.VMEM((n,), jnp.float32),
            pltpu.VMEM((n,), jnp.float32),
            pltpu.VMEM((_CHUNK,), jnp.int32),
            pltpu.VMEM((_CHUNK,), jnp.int32),
            pltpu.VMEM((_CHUNK,), jnp.float32),
            pltpu.VMEM((_CHUNK,), jnp.float32),
            pltpu.VMEM((_CHUNK, 128), jnp.float32),
            pltpu.VMEM((_L,), jnp.float32),
            pltpu.VMEM_SHARED((npad, 128), jnp.float32),
            pltpu.SemaphoreType.DMA,
        ],
        compiler_params=pltpu.CompilerParams(needs_layout_passes=False),
    )
    def edge_kernel(h_hbm, asrc_hbm, adst_hbm, src_hbm, dst_hbm, ea_hbm, ce_hbm,
                    num_out, den_out,
                    asrc_v, adst_v, den_v, srcv, dstv, eav, sv, rows, cev,
                    num_sh, sem):
        cid = lax.axis_index("c")
        sid = lax.axis_index("s")
        wid = cid * _NSUB + sid

        pltpu.sync_copy(asrc_hbm, asrc_v)
        pltpu.sync_copy(adst_hbm, adst_v)
        pltpu.sync_copy(ce_hbm, cev)
        cv = cev[...]

        zf = jnp.zeros((_L,), jnp.float32)

        def _zden(i, carry):
            den_v[pl.ds(i * _L, _L)] = zf
            return carry
        lax.fori_loop(0, n // _L, _zden, 0)

        def _zrows(r, carry):
            for q in range(8):
                rows[r, pl.ds(q * _L, _L)] = zf
            return carry
        lax.fori_loop(0, _CHUNK, _zrows, 0)

        base_sh = pl.multiple_of(sid * rows_per_tile, _CHUNK)
        for t in range(nfull):
            pltpu.sync_copy(rows, num_sh.at[pl.ds(base_sh + t * _CHUNK, _CHUNK)])
        plsc.subcore_barrier()

        ebase = wid * ept

        def _chunk(c, carry):
            base = ebase + c * _CHUNK
            pltpu.sync_copy(src_hbm.at[pl.ds(base, _CHUNK)], srcv)
            pltpu.sync_copy(dst_hbm.at[pl.ds(base, _CHUNK)], dstv)
            pltpu.sync_copy(ea_hbm.at[pl.ds(base, _CHUNK)], eav)
            cp = pltpu.async_copy(h_hbm.at[srcv], rows, sem)

            def _sbody(j, carry2):
                si = srcv[pl.ds(j * _L, _L)]
                di = dstv[pl.ds(j * _L, _L)]
                av = plsc.load_gather(asrc_v, [si])
                bv = plsc.load_gather(adst_v, [di])
                al = av + bv + eav[pl.ds(j * _L, _L)] * cv
                al = jnp.maximum(al, 0.2 * al)
                sval = jnp.exp(al)
                gid = base + j * _L + lax.iota(jnp.int32, _L)
                sval = jnp.where(gid < e, sval, 0.0)
                sv[pl.ds(j * _L, _L)] = sval
                plsc.addupdate_scatter(den_v, [di], sval)
                return carry2
            lax.fori_loop(0, _CHUNK // _L, _sbody, 0)

            cp.wait()

            def _scale(r, carry2):
                sb = plsc.load_gather(sv, [jnp.broadcast_to(r, (_L,))])
                for q in range(8):
                    rows[r, pl.ds(q * _L, _L)] = rows[r, pl.ds(q * _L, _L)] * sb
                return carry2
            lax.fori_loop(0, _CHUNK, _scale, 0)

            pltpu.sync_copy(rows, num_sh.at[dstv], add=True)
            return carry
        lax.fori_loop(0, nchunk, _chunk, 0)

        pltpu.sync_copy(den_v, den_out.at[pl.ds(pl.multiple_of(wid * n, 8), n)])
        plsc.subcore_barrier()

        obase = pl.multiple_of(cid * npad + base_sh, _CHUNK)
        for t in range(nfull):
            pltpu.sync_copy(num_sh.at[pl.ds(base_sh + t * _CHUNK, _CHUNK)],
                            num_out.at[pl.ds(obase + t * _CHUNK, _CHUNK)])

    return edge_kernel


def kernel(x, edge_index, edge_attr, W1, att_src1, att_dst1, W_e1, att_e1, b1,
           W2, att_src2, att_dst2, W_e2, att_e2, b2):
    n = x.shape[0]
    e = edge_index.shape[1]

    src = edge_index[0].astype(jnp.int32)
    dst = edge_index[1].astype(jnp.int32)
    ea = edge_attr[:, 0].astype(jnp.float32)

    ept = -(-e // (_NW * _CHUNK)) * _CHUNK
    pad = _NW * ept - e
    src_p = jnp.pad(src, (0, pad))
    dst_p = jnp.pad(dst, (0, pad))
    ea_p = jnp.pad(ea, (0, pad))

    npad = -(-n // (_NSUB * _CHUNK)) * (_NSUB * _CHUNK)
    edge_kernel = _make_edge_kernel(n, e, ept)

    h1, s1, d1, ce1 = _run_dense1(
        x, W1, att_src1.reshape(128, 1), att_dst1.reshape(128, 1),
        W_e1, att_e1.reshape(1, 128), n)
    num1, den1 = edge_kernel(
        h1, s1.reshape(n), d1.reshape(n), src_p, dst_p, ea_p, ce1[0, :_L])

    h2, s2, d2, ce2 = _run_mid(
        num1.reshape(2, npad, 128), den1.reshape(_NW, n).T, b1.reshape(1, 128),
        W2, att_src2.reshape(128, 1), att_dst2.reshape(128, 1),
        W_e2, att_e2.reshape(1, 128), n)
    num2, den2 = edge_kernel(
        h2, s2.reshape(n), d2.reshape(n), src_p, dst_p, ea_p, ce2[0, :_L])

    out = _run_final(num2.reshape(2, npad, 128), den2.reshape(_NW, n).T,
                     b2.reshape(1, 128), n)
    return out


# speedup vs baseline: 21.9969x; 1.2241x over previous
import functools

import jax
import jax.numpy as jnp
from jax import lax
from jax.experimental import pallas as pl
from jax.experimental.pallas import tpu as pltpu
from jax.experimental.pallas import tpu_sc as plsc

_L = 16
_NSUB = 16
_NCORE = 2
_NW = _NSUB * _NCORE
_CHUNK = 64
_BLK = 2000


def _dense1_body(x_ref, w_ref, asr_ref, adr_ref, we_ref, ate_ref,
                 h_ref, s_ref, d_ref, ce_ref):
    h = jnp.dot(x_ref[...], w_ref[...], preferred_element_type=jnp.float32)
    h_ref[...] = h
    s_ref[...] = jnp.dot(h, asr_ref[...], preferred_element_type=jnp.float32)
    d_ref[...] = jnp.dot(h, adr_ref[...], preferred_element_type=jnp.float32)
    ce_ref[...] = jnp.full((1, 128), jnp.sum(we_ref[...] * ate_ref[...]),
                           dtype=jnp.float32)


def _mid_body(num_ref, den_ref, b_ref, w_ref, asr_ref, adr_ref, we_ref, ate_ref,
              h_ref, s_ref, d_ref, ce_ref):
    nm = num_ref[0] + num_ref[1]
    dn = jnp.sum(den_ref[...], axis=1, keepdims=True)
    h1 = nm / (dn + 1e-16) + b_ref[...]
    h1 = jnp.maximum(h1, 0.0)
    h = jnp.dot(h1, w_ref[...], preferred_element_type=jnp.float32)
    h_ref[...] = h
    s_ref[...] = jnp.dot(h, asr_ref[...], preferred_element_type=jnp.float32)
    d_ref[...] = jnp.dot(h, adr_ref[...], preferred_element_type=jnp.float32)
    ce_ref[...] = jnp.full((1, 128), jnp.sum(we_ref[...] * ate_ref[...]),
                           dtype=jnp.float32)


def _final_body(num_ref, den_ref, b_ref, out_ref):
    nm = num_ref[0] + num_ref[1]
    dn = jnp.sum(den_ref[...], axis=1, keepdims=True)
    out_ref[...] = nm / (dn + 1e-16) + b_ref[...]


def _run_dense1(x, W, att_s, att_d, We, att_e, n):
    grid = n // _BLK
    full = lambda i: (0, 0)
    row = lambda i: (i, 0)
    return pl.pallas_call(
        _dense1_body,
        grid=(grid,),
        in_specs=[
            pl.BlockSpec((_BLK, 128), row),
            pl.BlockSpec((128, 128), full),
            pl.BlockSpec((128, 1), full),
            pl.BlockSpec((128, 1), full),
            pl.BlockSpec((1, 128), full),
            pl.BlockSpec((1, 128), full),
        ],
        out_specs=[
            pl.BlockSpec((_BLK, 128), row),
            pl.BlockSpec((_BLK, 1), row),
            pl.BlockSpec((_BLK, 1), row),
            pl.BlockSpec((1, 128), full),
        ],
        out_shape=[
            jax.ShapeDtypeStruct((n, 128), jnp.float32),
            jax.ShapeDtypeStruct((n, 1), jnp.float32),
            jax.ShapeDtypeStruct((n, 1), jnp.float32),
            jax.ShapeDtypeStruct((1, 128), jnp.float32),
        ],
    )(x, W, att_s, att_d, We, att_e)


def _run_mid(num, den_t, b, W, att_s, att_d, We, att_e, n):
    grid = n // _BLK
    full = lambda i: (0, 0)
    row = lambda i: (i, 0)
    return pl.pallas_call(
        _mid_body,
        grid=(grid,),
        in_specs=[
            pl.BlockSpec((2, _BLK, 128), lambda i: (0, i, 0)),
            pl.BlockSpec((_BLK, 32), row),
            pl.BlockSpec((1, 128), full),
            pl.BlockSpec((128, 128), full),
            pl.BlockSpec((128, 1), full),
            pl.BlockSpec((128, 1), full),
            pl.BlockSpec((1, 128), full),
            pl.BlockSpec((1, 128), full),
        ],
        out_specs=[
            pl.BlockSpec((_BLK, 128), row),
            pl.BlockSpec((_BLK, 1), row),
            pl.BlockSpec((_BLK, 1), row),
            pl.BlockSpec((1, 128), full),
        ],
        out_shape=[
            jax.ShapeDtypeStruct((n, 128), jnp.float32),
            jax.ShapeDtypeStruct((n, 1), jnp.float32),
            jax.ShapeDtypeStruct((n, 1), jnp.float32),
            jax.ShapeDtypeStruct((1, 128), jnp.float32),
        ],
    )(num, den_t, b, W, att_s, att_d, We, att_e)


def _run_final(num, den_t, b, n):
    grid = n // _BLK
    full = lambda i: (0, 0)
    row = lambda i: (i, 0)
    return pl.pallas_call(
        _final_body,
        grid=(grid,),
        in_specs=[
            pl.BlockSpec((2, _BLK, 128), lambda i: (0, i, 0)),
            pl.BlockSpec((_BLK, 32), row),
            pl.BlockSpec((1, 128), full),
        ],
        out_specs=pl.BlockSpec((_BLK, 128), row),
        out_shape=jax.ShapeDtypeStruct((n, 128), jnp.float32),
    )(num, den_t, b)


def _make_edge_kernel(n, e, ept):
    nchunk = ept // _CHUNK
    npad = -(-n // (_NSUB * _CHUNK)) * (_NSUB * _CHUNK)
    rows_per_tile = npad // _NSUB
    nfull = rows_per_tile // _CHUNK

    mesh = plsc.VectorSubcoreMesh(core_axis_name="c", subcore_axis_name="s")

    @functools.partial(
        pl.kernel,
        out_type=[
            jax.ShapeDtypeStruct((_NCORE * npad, 128), jnp.float32),
            jax.ShapeDtypeStruct((_NW * n,), jnp.float32),
        ],
        mesh=mesh,
        scratch_types=[
            pltpu.VMEM((n,), jnp.float32),
            pltpu.VMEM((n,), jnp.float32),
            pltpu.VMEM((n,), jnp.float32),
            [pltpu.VMEM((_CHUNK,), jnp.int32) for _ in range(2)],
            [pltpu.VMEM((_CHUNK,), jnp.int32) for _ in range(2)],
            [pltpu.VMEM((_CHUNK,), jnp.float32) for _ in range(2)],
            [pltpu.VMEM((_CHUNK,), jnp.int32) for _ in range(2)],
            [pltpu.VMEM((_CHUNK, 128), jnp.float32) for _ in range(2)],
            pltpu.VMEM((_CHUNK,), jnp.float32),
            pltpu.VMEM((_L,), jnp.float32),
            pltpu.VMEM_SHARED((npad, 128), jnp.float32),
            [pltpu.SemaphoreType.DMA for _ in range(6)],
        ],
        compiler_params=pltpu.CompilerParams(needs_layout_passes=False),
    )
    def edge_kernel(h_hbm, asrc_hbm, adst_hbm, src_hbm, dst_hbm, ea_hbm, ce_hbm,
                    num_out, den_out,
                    asrc_v, adst_v, den_v, srcv, dstv, eav, scidx, rows, sv,
                    cev, num_sh, sems):
        cid = lax.axis_index("c")
        sid = lax.axis_index("s")
        wid = cid * _NSUB + sid
        semI = sems[0:2]
        semG = sems[2:4]
        semS = sems[4:6]

        pltpu.sync_copy(asrc_hbm, asrc_v)
        pltpu.sync_copy(adst_hbm, adst_v)
        pltpu.sync_copy(ce_hbm, cev)
        cv = cev[...]

        zf = jnp.zeros((_L,), jnp.float32)

        def _zden(i, carry):
            den_v[pl.ds(i * _L, _L)] = zf
            return carry
        lax.fori_loop(0, n // _L, _zden, 0)

        def _zrows(r, carry):
            for q in range(8):
                rows[0][r, pl.ds(q * _L, _L)] = zf
            return carry
        lax.fori_loop(0, _CHUNK, _zrows, 0)

        base_sh = pl.multiple_of(sid * rows_per_tile, _CHUNK)
        for t in range(nfull):
            pltpu.sync_copy(rows[0],
                            num_sh.at[pl.ds(base_sh + t * _CHUNK, _CHUNK)])
        plsc.subcore_barrier()

        ebase = wid * ept

        def _start_idx(c, b):
            base = ebase + c * _CHUNK
            pltpu.async_copy(src_hbm.at[pl.ds(base, _CHUNK)], srcv[b], semI[b])
            pltpu.async_copy(dst_hbm.at[pl.ds(base, _CHUNK)], dstv[b], semI[b])
            pltpu.async_copy(ea_hbm.at[pl.ds(base, _CHUNK)], eav[b], semI[b])

        def _wait_idx(c, b):
            base = ebase + c * _CHUNK
            pltpu.make_async_copy(src_hbm.at[pl.ds(base, _CHUNK)], srcv[b],
                                  semI[b]).wait()
            pltpu.make_async_copy(dst_hbm.at[pl.ds(base, _CHUNK)], dstv[b],
                                  semI[b]).wait()
            pltpu.make_async_copy(ea_hbm.at[pl.ds(base, _CHUNK)], eav[b],
                                  semI[b]).wait()

        def _wait_scatter(b):
            pltpu.make_async_copy(rows[b], num_sh.at[scidx[b]], semS[b]).wait()

        _start_idx(0, 0)
        _start_idx(1, 1)

        def _chunk_pair(t, carry):
            for b in range(2):
                c = 2 * t + b
                base = ebase + c * _CHUNK
                _wait_idx(c, b)
                @pl.when(t > 0)
                def _():
                    _wait_scatter(b)
                pltpu.async_copy(h_hbm.at[srcv[b]], rows[b], semG[b])

                def _sbody(j, carry2):
                    si = srcv[b][pl.ds(j * _L, _L)]
                    di = dstv[b][pl.ds(j * _L, _L)]
                    av = plsc.load_gather(asrc_v, [si])
                    bv = plsc.load_gather(adst_v, [di])
                    al = av + bv + eav[b][pl.ds(j * _L, _L)] * cv
                    al = jnp.maximum(al, 0.2 * al)
                    sval = jnp.exp(al)
                    gid = base + j * _L + lax.iota(jnp.int32, _L)
                    sval = jnp.where(gid < e, sval, 0.0)
                    sv[pl.ds(j * _L, _L)] = sval
                    plsc.addupdate_scatter(den_v, [di], sval)
                    return carry2
                lax.fori_loop(0, _CHUNK // _L, _sbody, 0)

                for q in range(_CHUNK // _L):
                    scidx[b][pl.ds(q * _L, _L)] = dstv[b][pl.ds(q * _L, _L)]

                pltpu.make_async_copy(h_hbm.at[srcv[b]], rows[b],
                                      semG[b]).wait()

                def _scale(i, carry2):
                    for dr in range(4):
                        r = i * 4 + dr
                        sb = plsc.load_gather(sv, [jnp.broadcast_to(r, (_L,))])
                        for q in range(8):
                            rows[b][r, pl.ds(q * _L, _L)] = (
                                rows[b][r, pl.ds(q * _L, _L)] * sb)
                    return carry2
                lax.fori_loop(0, _CHUNK // 4, _scale, 0)

                pltpu.async_copy(rows[b], num_sh.at[scidx[b]], semS[b],
                                 add=True)

                @pl.when(c + 2 < nchunk)
                def _():
                    _start_idx(c + 2, b)
            return carry
        lax.fori_loop(0, nchunk // 2, _chunk_pair, 0)
        _wait_scatter(0)
        _wait_scatter(1)

        pltpu.sync_copy(den_v, den_out.at[pl.ds(pl.multiple_of(wid * n, 8), n)])
        plsc.subcore_barrier()

        obase = pl.multiple_of(cid * npad + base_sh, _CHUNK)
        for t in range(nfull):
            pltpu.sync_copy(num_sh.at[pl.ds(base_sh + t * _CHUNK, _CHUNK)],
                            num_out.at[pl.ds(obase + t * _CHUNK, _CHUNK)])

    return edge_kernel


def kernel(x, edge_index, edge_attr, W1, att_src1, att_dst1, W_e1, att_e1, b1,
           W2, att_src2, att_dst2, W_e2, att_e2, b2):
    n = x.shape[0]
    e = edge_index.shape[1]

    src = edge_index[0].astype(jnp.int32)
    dst = edge_index[1].astype(jnp.int32)
    ea = edge_attr[:, 0].astype(jnp.float32)

    ept = -(-e // (_NW * 2 * _CHUNK)) * (2 * _CHUNK)
    pad = _NW * ept - e
    src_p = jnp.pad(src, (0, pad))
    dst_p = jnp.pad(dst, (0, pad))
    ea_p = jnp.pad(ea, (0, pad))

    npad = -(-n // (_NSUB * _CHUNK)) * (_NSUB * _CHUNK)
    edge_kernel = _make_edge_kernel(n, e, ept)

    h1, s1, d1, ce1 = _run_dense1(
        x, W1, att_src1.reshape(128, 1), att_dst1.reshape(128, 1),
        W_e1, att_e1.reshape(1, 128), n)
    num1, den1 = edge_kernel(
        h1, s1.reshape(n), d1.reshape(n), src_p, dst_p, ea_p, ce1[0, :_L])

    h2, s2, d2, ce2 = _run_mid(
        num1.reshape(2, npad, 128), den1.reshape(_NW, n).T, b1.reshape(1, 128),
        W2, att_src2.reshape(128, 1), att_dst2.reshape(128, 1),
        W_e2, att_e2.reshape(1, 128), n)
    num2, den2 = edge_kernel(
        h2, s2.reshape(n), d2.reshape(n), src_p, dst_p, ea_p, ce2[0, :_L])

    out = _run_final(num2.reshape(2, npad, 128), den2.reshape(_NW, n).T,
                     b2.reshape(1, 128), n)
    return out


# speedup vs baseline: 26.1853x; 1.1904x over previous
import functools

import jax
import jax.numpy as jnp
from jax import lax
from jax.experimental import pallas as pl
from jax.experimental.pallas import tpu as pltpu
from jax.experimental.pallas import tpu_sc as plsc

_L = 16
_NSUB = 16
_NCORE = 2
_NW = _NSUB * _NCORE
_CHUNK = 64
_BLK = 2000


def _dense1_body(x_ref, w_ref, asr_ref, adr_ref, we_ref, ate_ref,
                 h_ref, s_ref, d_ref, ce_ref):
    h = jnp.dot(x_ref[...], w_ref[...], preferred_element_type=jnp.float32)
    h_ref[...] = h
    s_ref[...] = jnp.dot(h, asr_ref[...], preferred_element_type=jnp.float32)
    d_ref[...] = jnp.dot(h, adr_ref[...], preferred_element_type=jnp.float32)
    ce_ref[...] = jnp.full((1, 128), jnp.sum(we_ref[...] * ate_ref[...]),
                           dtype=jnp.float32)


def _mid_body(num_ref, den_ref, b_ref, w_ref, asr_ref, adr_ref, we_ref, ate_ref,
              h_ref, s_ref, d_ref, ce_ref):
    nm = num_ref[0] + num_ref[1]
    dn = jnp.sum(den_ref[...], axis=1, keepdims=True)
    h1 = nm / (dn + 1e-16) + b_ref[...]
    h1 = jnp.maximum(h1, 0.0)
    h = jnp.dot(h1, w_ref[...], preferred_element_type=jnp.float32)
    h_ref[...] = h
    s_ref[...] = jnp.dot(h, asr_ref[...], preferred_element_type=jnp.float32)
    d_ref[...] = jnp.dot(h, adr_ref[...], preferred_element_type=jnp.float32)
    ce_ref[...] = jnp.full((1, 128), jnp.sum(we_ref[...] * ate_ref[...]),
                           dtype=jnp.float32)


def _final_body(num_ref, den_ref, b_ref, out_ref):
    nm = num_ref[0] + num_ref[1]
    dn = jnp.sum(den_ref[...], axis=1, keepdims=True)
    out_ref[...] = nm / (dn + 1e-16) + b_ref[...]


def _run_dense1(x, W, att_s, att_d, We, att_e, n):
    grid = n // _BLK
    full = lambda i: (0, 0)
    row = lambda i: (i, 0)
    return pl.pallas_call(
        _dense1_body,
        grid=(grid,),
        in_specs=[
            pl.BlockSpec((_BLK, 128), row),
            pl.BlockSpec((128, 128), full),
            pl.BlockSpec((128, 1), full),
            pl.BlockSpec((128, 1), full),
            pl.BlockSpec((1, 128), full),
            pl.BlockSpec((1, 128), full),
        ],
        out_specs=[
            pl.BlockSpec((_BLK, 128), row),
            pl.BlockSpec((_BLK, 1), row),
            pl.BlockSpec((_BLK, 1), row),
            pl.BlockSpec((1, 128), full),
        ],
        out_shape=[
            jax.ShapeDtypeStruct((n, 128), jnp.float32),
            jax.ShapeDtypeStruct((n, 1), jnp.float32),
            jax.ShapeDtypeStruct((n, 1), jnp.float32),
            jax.ShapeDtypeStruct((1, 128), jnp.float32),
        ],
    )(x, W, att_s, att_d, We, att_e)


def _run_mid(num, den_t, b, W, att_s, att_d, We, att_e, n):
    grid = n // _BLK
    full = lambda i: (0, 0)
    row = lambda i: (i, 0)
    return pl.pallas_call(
        _mid_body,
        grid=(grid,),
        in_specs=[
            pl.BlockSpec((2, _BLK, 128), lambda i: (0, i, 0)),
            pl.BlockSpec((_BLK, 32), row),
            pl.BlockSpec((1, 128), full),
            pl.BlockSpec((128, 128), full),
            pl.BlockSpec((128, 1), full),
            pl.BlockSpec((128, 1), full),
            pl.BlockSpec((1, 128), full),
            pl.BlockSpec((1, 128), full),
        ],
        out_specs=[
            pl.BlockSpec((_BLK, 128), row),
            pl.BlockSpec((_BLK, 1), row),
            pl.BlockSpec((_BLK, 1), row),
            pl.BlockSpec((1, 128), full),
        ],
        out_shape=[
            jax.ShapeDtypeStruct((n, 128), jnp.float32),
            jax.ShapeDtypeStruct((n, 1), jnp.float32),
            jax.ShapeDtypeStruct((n, 1), jnp.float32),
            jax.ShapeDtypeStruct((1, 128), jnp.float32),
        ],
    )(num, den_t, b, W, att_s, att_d, We, att_e)


def _run_final(num, den_t, b, n):
    grid = n // _BLK
    full = lambda i: (0, 0)
    row = lambda i: (i, 0)
    return pl.pallas_call(
        _final_body,
        grid=(grid,),
        in_specs=[
            pl.BlockSpec((2, _BLK, 128), lambda i: (0, i, 0)),
            pl.BlockSpec((_BLK, 32), row),
            pl.BlockSpec((1, 128), full),
        ],
        out_specs=pl.BlockSpec((_BLK, 128), row),
        out_shape=jax.ShapeDtypeStruct((n, 128), jnp.float32),
    )(num, den_t, b)


def _make_edge_kernel(n, e, ept):
    nchunk = ept // _CHUNK
    npad = -(-n // (_NSUB * _CHUNK)) * (_NSUB * _CHUNK)
    rows_per_tile = npad // _NSUB
    nfull = rows_per_tile // _CHUNK

    mesh = plsc.VectorSubcoreMesh(core_axis_name="c", subcore_axis_name="s")

    @functools.partial(
        pl.kernel,
        out_type=[
            jax.ShapeDtypeStruct((_NCORE * npad, 128), jnp.float32),
            jax.ShapeDtypeStruct((_NW * n,), jnp.float32),
        ],
        mesh=mesh,
        scratch_types=[
            pltpu.VMEM((n,), jnp.float32),
            pltpu.VMEM((n,), jnp.float32),
            pltpu.VMEM((n,), jnp.float32),
            [pltpu.VMEM((_CHUNK,), jnp.int32) for _ in range(2)],
            [pltpu.VMEM((_CHUNK,), jnp.int32) for _ in range(2)],
            [pltpu.VMEM((_CHUNK,), jnp.float32) for _ in range(2)],
            [pltpu.VMEM((_CHUNK,), jnp.int32) for _ in range(2)],
            [pltpu.VMEM((_CHUNK, 128), jnp.float32) for _ in range(2)],
            [pltpu.VMEM((_CHUNK,), jnp.float32) for _ in range(2)],
            pltpu.VMEM((_L,), jnp.float32),
            pltpu.VMEM_SHARED((npad, 128), jnp.float32),
            [pltpu.SemaphoreType.DMA for _ in range(6)],
        ],
        compiler_params=pltpu.CompilerParams(needs_layout_passes=False),
    )
    def edge_kernel(h_hbm, asrc_hbm, adst_hbm, src_hbm, dst_hbm, ea_hbm, ce_hbm,
                    num_out, den_out,
                    asrc_v, adst_v, den_v, srcv, dstv, eav, scidx, rows, sv,
                    cev, num_sh, sems):
        cid = lax.axis_index("c")
        sid = lax.axis_index("s")
        wid = cid * _NSUB + sid
        semI = sems[0:2]
        semG = sems[2:4]
        semS = sems[4:6]

        pltpu.sync_copy(asrc_hbm, asrc_v)
        pltpu.sync_copy(adst_hbm, adst_v)
        pltpu.sync_copy(ce_hbm, cev)
        cv = cev[...]

        zf = jnp.zeros((_L,), jnp.float32)

        def _zden(i, carry):
            den_v[pl.ds(i * _L, _L)] = zf
            return carry
        lax.fori_loop(0, n // _L, _zden, 0)

        def _zrows(r, carry):
            for q in range(8):
                rows[0][r, pl.ds(q * _L, _L)] = zf
            return carry
        lax.fori_loop(0, _CHUNK, _zrows, 0)

        base_sh = pl.multiple_of(sid * rows_per_tile, _CHUNK)
        for t in range(nfull):
            pltpu.sync_copy(rows[0],
                            num_sh.at[pl.ds(base_sh + t * _CHUNK, _CHUNK)])
        plsc.subcore_barrier()

        ebase = wid * ept

        def _start_idx(c, b):
            base = ebase + c * _CHUNK
            pltpu.async_copy(src_hbm.at[pl.ds(base, _CHUNK)], srcv[b], semI[b])
            pltpu.async_copy(dst_hbm.at[pl.ds(base, _CHUNK)], dstv[b], semI[b])
            pltpu.async_copy(ea_hbm.at[pl.ds(base, _CHUNK)], eav[b], semI[b])

        def _wait_idx(c, b):
            base = ebase + c * _CHUNK
            pltpu.make_async_copy(src_hbm.at[pl.ds(base, _CHUNK)], srcv[b],
                                  semI[b]).wait()
            pltpu.make_async_copy(dst_hbm.at[pl.ds(base, _CHUNK)], dstv[b],
                                  semI[b]).wait()
            pltpu.make_async_copy(ea_hbm.at[pl.ds(base, _CHUNK)], eav[b],
                                  semI[b]).wait()

        def _wait_scatter(b):
            pltpu.make_async_copy(rows[b], num_sh.at[scidx[b]], semS[b]).wait()

        _start_idx(0, 0)
        _start_idx(1, 1)

        def _chunk_pair(t, carry):
            c0 = 2 * t
            for b in range(2):
                _wait_idx(c0 + b, b)
                @pl.when(t > 0)
                def _():
                    _wait_scatter(b)
                pltpu.async_copy(h_hbm.at[srcv[b]], rows[b], semG[b])

            for b in range(2):
                base = ebase + (c0 + b) * _CHUNK

                def _sbody(j, carry2):
                    si = srcv[b][pl.ds(j * _L, _L)]
                    di = dstv[b][pl.ds(j * _L, _L)]
                    av = plsc.load_gather(asrc_v, [si])
                    bv = plsc.load_gather(adst_v, [di])
                    al = av + bv + eav[b][pl.ds(j * _L, _L)] * cv
                    al = jnp.maximum(al, 0.2 * al)
                    sval = jnp.exp(al)
                    gid = base + j * _L + lax.iota(jnp.int32, _L)
                    sval = jnp.where(gid < e, sval, 0.0)
                    sv[b][pl.ds(j * _L, _L)] = sval
                    plsc.addupdate_scatter(den_v, [di], sval)
                    return carry2
                lax.fori_loop(0, _CHUNK // _L, _sbody, 0)

                for q in range(_CHUNK // _L):
                    scidx[b][pl.ds(q * _L, _L)] = dstv[b][pl.ds(q * _L, _L)]

            for b in range(2):
                pltpu.make_async_copy(h_hbm.at[srcv[b]], rows[b],
                                      semG[b]).wait()

                def _scale(i, carry2):
                    for dr in range(8):
                        r = i * 8 + dr
                        sb = plsc.load_gather(sv[b],
                                              [jnp.broadcast_to(r, (_L,))])
                        for q in range(8):
                            rows[b][r, pl.ds(q * _L, _L)] = (
                                rows[b][r, pl.ds(q * _L, _L)] * sb)
                    return carry2
                lax.fori_loop(0, _CHUNK // 8, _scale, 0)

                pltpu.async_copy(rows[b], num_sh.at[scidx[b]], semS[b],
                                 add=True)

                @pl.when(c0 + b + 2 < nchunk)
                def _():
                    _start_idx(c0 + b + 2, b)
            return carry
        lax.fori_loop(0, nchunk // 2, _chunk_pair, 0)
        _wait_scatter(0)
        _wait_scatter(1)

        pltpu.sync_copy(den_v, den_out.at[pl.ds(pl.multiple_of(wid * n, 8), n)])
        plsc.subcore_barrier()

        obase = pl.multiple_of(cid * npad + base_sh, _CHUNK)
        for t in range(nfull):
            pltpu.sync_copy(num_sh.at[pl.ds(base_sh + t * _CHUNK, _CHUNK)],
                            num_out.at[pl.ds(obase + t * _CHUNK, _CHUNK)])

    return edge_kernel


def kernel(x, edge_index, edge_attr, W1, att_src1, att_dst1, W_e1, att_e1, b1,
           W2, att_src2, att_dst2, W_e2, att_e2, b2):
    n = x.shape[0]
    e = edge_index.shape[1]

    src = edge_index[0].astype(jnp.int32)
    dst = edge_index[1].astype(jnp.int32)
    ea = edge_attr[:, 0].astype(jnp.float32)

    ept = -(-e // (_NW * 2 * _CHUNK)) * (2 * _CHUNK)
    pad = _NW * ept - e
    src_p = jnp.pad(src, (0, pad))
    dst_p = jnp.pad(dst, (0, pad))
    ea_p = jnp.pad(ea, (0, pad))

    npad = -(-n // (_NSUB * _CHUNK)) * (_NSUB * _CHUNK)
    edge_kernel = _make_edge_kernel(n, e, ept)

    h1, s1, d1, ce1 = _run_dense1(
        x, W1, att_src1.reshape(128, 1), att_dst1.reshape(128, 1),
        W_e1, att_e1.reshape(1, 128), n)
    num1, den1 = edge_kernel(
        h1, s1.reshape(n), d1.reshape(n), src_p, dst_p, ea_p, ce1[0, :_L])

    h2, s2, d2, ce2 = _run_mid(
        num1.reshape(2, npad, 128), den1.reshape(_NW, n).T, b1.reshape(1, 128),
        W2, att_src2.reshape(128, 1), att_dst2.reshape(128, 1),
        W_e2, att_e2.reshape(1, 128), n)
    num2, den2 = edge_kernel(
        h2, s2.reshape(n), d2.reshape(n), src_p, dst_p, ea_p, ce2[0, :_L])

    out = _run_final(num2.reshape(2, npad, 128), den2.reshape(_NW, n).T,
                     b2.reshape(1, 128), n)
    return out


# speedup vs baseline: 27.6033x; 1.0542x over previous
import functools

import jax
import jax.numpy as jnp
from jax import lax
from jax.experimental import pallas as pl
from jax.experimental.pallas import tpu as pltpu
from jax.experimental.pallas import tpu_sc as plsc

_L = 16
_NSUB = 16
_NCORE = 2
_NW = _NSUB * _NCORE
_CHUNK = 64
_BLK = 2000


def _dense1_body(x_ref, w_ref, asr_ref, adr_ref, we_ref, ate_ref,
                 h_ref, s_ref, d_ref, ce_ref):
    h = jnp.dot(x_ref[...], w_ref[...], preferred_element_type=jnp.float32)
    h_ref[...] = h
    s_ref[...] = jnp.dot(h, asr_ref[...], preferred_element_type=jnp.float32)
    d_ref[...] = jnp.dot(h, adr_ref[...], preferred_element_type=jnp.float32)
    ce_ref[...] = jnp.full((1, 128), jnp.sum(we_ref[...] * ate_ref[...]),
                           dtype=jnp.float32)


def _mid_body(num_ref, den_ref, b_ref, w_ref, asr_ref, adr_ref, we_ref, ate_ref,
              h_ref, s_ref, d_ref, ce_ref):
    nm = num_ref[0] + num_ref[1]
    dn = jnp.sum(den_ref[...], axis=1, keepdims=True)
    h1 = nm / (dn + 1e-16) + b_ref[...]
    h1 = jnp.maximum(h1, 0.0)
    h = jnp.dot(h1, w_ref[...], preferred_element_type=jnp.float32)
    h_ref[...] = h
    s_ref[...] = jnp.dot(h, asr_ref[...], preferred_element_type=jnp.float32)
    d_ref[...] = jnp.dot(h, adr_ref[...], preferred_element_type=jnp.float32)
    ce_ref[...] = jnp.full((1, 128), jnp.sum(we_ref[...] * ate_ref[...]),
                           dtype=jnp.float32)


def _final_body(num_ref, den_ref, b_ref, out_ref):
    nm = num_ref[0] + num_ref[1]
    dn = jnp.sum(den_ref[...], axis=1, keepdims=True)
    out_ref[...] = nm / (dn + 1e-16) + b_ref[...]


def _run_dense1(x, W, att_s, att_d, We, att_e, n):
    grid = n // _BLK
    full = lambda i: (0, 0)
    row = lambda i: (i, 0)
    return pl.pallas_call(
        _dense1_body,
        grid=(grid,),
        in_specs=[
            pl.BlockSpec((_BLK, 128), row),
            pl.BlockSpec((128, 128), full),
            pl.BlockSpec((128, 1), full),
            pl.BlockSpec((128, 1), full),
            pl.BlockSpec((1, 128), full),
            pl.BlockSpec((1, 128), full),
        ],
        out_specs=[
            pl.BlockSpec((_BLK, 128), row),
            pl.BlockSpec((_BLK, 1), row),
            pl.BlockSpec((_BLK, 1), row),
            pl.BlockSpec((1, 128), full),
        ],
        out_shape=[
            jax.ShapeDtypeStruct((n, 128), jnp.float32),
            jax.ShapeDtypeStruct((n, 1), jnp.float32),
            jax.ShapeDtypeStruct((n, 1), jnp.float32),
            jax.ShapeDtypeStruct((1, 128), jnp.float32),
        ],
    )(x, W, att_s, att_d, We, att_e)


def _run_mid(num, den_t, b, W, att_s, att_d, We, att_e, n):
    grid = n // _BLK
    full = lambda i: (0, 0)
    row = lambda i: (i, 0)
    return pl.pallas_call(
        _mid_body,
        grid=(grid,),
        in_specs=[
            pl.BlockSpec((2, _BLK, 128), lambda i: (0, i, 0)),
            pl.BlockSpec((_BLK, 32), row),
            pl.BlockSpec((1, 128), full),
            pl.BlockSpec((128, 128), full),
            pl.BlockSpec((128, 1), full),
            pl.BlockSpec((128, 1), full),
            pl.BlockSpec((1, 128), full),
            pl.BlockSpec((1, 128), full),
        ],
        out_specs=[
            pl.BlockSpec((_BLK, 128), row),
            pl.BlockSpec((_BLK, 1), row),
            pl.BlockSpec((_BLK, 1), row),
            pl.BlockSpec((1, 128), full),
        ],
        out_shape=[
            jax.ShapeDtypeStruct((n, 128), jnp.float32),
            jax.ShapeDtypeStruct((n, 1), jnp.float32),
            jax.ShapeDtypeStruct((n, 1), jnp.float32),
            jax.ShapeDtypeStruct((1, 128), jnp.float32),
        ],
    )(num, den_t, b, W, att_s, att_d, We, att_e)


def _run_final(num, den_t, b, n):
    grid = n // _BLK
    full = lambda i: (0, 0)
    row = lambda i: (i, 0)
    return pl.pallas_call(
        _final_body,
        grid=(grid,),
        in_specs=[
            pl.BlockSpec((2, _BLK, 128), lambda i: (0, i, 0)),
            pl.BlockSpec((_BLK, 32), row),
            pl.BlockSpec((1, 128), full),
        ],
        out_specs=pl.BlockSpec((_BLK, 128), row),
        out_shape=jax.ShapeDtypeStruct((n, 128), jnp.float32),
    )(num, den_t, b)


def _make_edge_kernel(n, e, ept):
    nchunk = ept // _CHUNK
    npad = -(-n // (_NSUB * _CHUNK)) * (_NSUB * _CHUNK)
    rows_per_tile = npad // _NSUB
    nfull = rows_per_tile // _CHUNK

    mesh = plsc.VectorSubcoreMesh(core_axis_name="c", subcore_axis_name="s")

    @functools.partial(
        pl.kernel,
        out_type=[
            jax.ShapeDtypeStruct((_NCORE * npad, 128), jnp.float32),
            jax.ShapeDtypeStruct((_NW * n,), jnp.float32),
        ],
        mesh=mesh,
        scratch_types=[
            pltpu.VMEM((n,), jnp.float32),
            pltpu.VMEM((n,), jnp.float32),
            pltpu.VMEM((n,), jnp.float32),
            [pltpu.VMEM((_CHUNK,), jnp.int32) for _ in range(2)],
            [pltpu.VMEM((_CHUNK,), jnp.int32) for _ in range(2)],
            [pltpu.VMEM((_CHUNK,), jnp.float32) for _ in range(2)],
            [pltpu.VMEM((_CHUNK,), jnp.int32) for _ in range(2)],
            [pltpu.VMEM((_CHUNK, 128), jnp.float32) for _ in range(2)],
            [pltpu.VMEM((_CHUNK,), jnp.float32) for _ in range(2)],
            pltpu.VMEM((_L,), jnp.float32),
            pltpu.VMEM_SHARED((npad, 128), jnp.float32),
            [pltpu.SemaphoreType.DMA for _ in range(6)],
        ],
        compiler_params=pltpu.CompilerParams(needs_layout_passes=False),
    )
    def edge_kernel(h_hbm, asrc_hbm, adst_hbm, src_hbm, dst_hbm, ea_hbm, ce_hbm,
                    num_out, den_out,
                    asrc_v, adst_v, den_v, srcv, dstv, eav, scidx, rows, sv,
                    cev, num_sh, sems):
        cid = lax.axis_index("c")
        sid = lax.axis_index("s")
        wid = cid * _NSUB + sid
        semI = sems[0:2]
        semG = sems[2:4]
        semS = sems[4:6]

        pltpu.sync_copy(asrc_hbm, asrc_v)
        pltpu.sync_copy(adst_hbm, adst_v)
        pltpu.sync_copy(ce_hbm, cev)
        cv = cev[...]

        zf = jnp.zeros((_L,), jnp.float32)

        def _zden(i, carry):
            den_v[pl.ds(i * _L, _L)] = zf
            return carry
        lax.fori_loop(0, n // _L, _zden, 0)

        def _zrows(r, carry):
            for q in range(8):
                rows[0][r, pl.ds(q * _L, _L)] = zf
            return carry
        lax.fori_loop(0, _CHUNK, _zrows, 0)

        base_sh = pl.multiple_of(sid * rows_per_tile, _CHUNK)
        for t in range(nfull):
            pltpu.sync_copy(rows[0],
                            num_sh.at[pl.ds(base_sh + t * _CHUNK, _CHUNK)])
        plsc.subcore_barrier()

        ebase = wid * ept

        def _start_idx(c, b):
            base = ebase + c * _CHUNK
            pltpu.async_copy(src_hbm.at[pl.ds(base, _CHUNK)], srcv[b], semI[b])
            pltpu.async_copy(dst_hbm.at[pl.ds(base, _CHUNK)], dstv[b], semI[b])
            pltpu.async_copy(ea_hbm.at[pl.ds(base, _CHUNK)], eav[b], semI[b])

        def _wait_idx(c, b):
            base = ebase + c * _CHUNK
            pltpu.make_async_copy(src_hbm.at[pl.ds(base, _CHUNK)], srcv[b],
                                  semI[b]).wait()
            pltpu.make_async_copy(dst_hbm.at[pl.ds(base, _CHUNK)], dstv[b],
                                  semI[b]).wait()
            pltpu.make_async_copy(ea_hbm.at[pl.ds(base, _CHUNK)], eav[b],
                                  semI[b]).wait()

        def _wait_scatter(b):
            pltpu.make_async_copy(rows[b], num_sh.at[scidx[b]], semS[b]).wait()

        _start_idx(0, 0)
        _start_idx(1, 1)

        def _chunk_pair(t, carry):
            c0 = 2 * t
            for b in range(2):
                _wait_idx(c0 + b, b)
                @pl.when(t > 0)
                def _():
                    _wait_scatter(b)
                pltpu.async_copy(h_hbm.at[srcv[b]], rows[b], semG[b])

            for b in range(2):
                base = ebase + (c0 + b) * _CHUNK

                @plsc.parallel_loop(0, _CHUNK // _L, unroll=4)
                def _sbody(j):
                    si = srcv[b][pl.ds(j * _L, _L)]
                    di = dstv[b][pl.ds(j * _L, _L)]
                    av = plsc.load_gather(asrc_v, [si])
                    bv = plsc.load_gather(adst_v, [di])
                    al = av + bv + eav[b][pl.ds(j * _L, _L)] * cv
                    al = jnp.maximum(al, 0.2 * al)
                    sval = jnp.exp(al)
                    gid = base + j * _L + lax.iota(jnp.int32, _L)
                    sval = jnp.where(gid < e, sval, 0.0)
                    sv[b][pl.ds(j * _L, _L)] = sval
                    plsc.addupdate_scatter(den_v, [di], sval)

                for q in range(_CHUNK // _L):
                    scidx[b][pl.ds(q * _L, _L)] = dstv[b][pl.ds(q * _L, _L)]

            for b in range(2):
                pltpu.make_async_copy(h_hbm.at[srcv[b]], rows[b],
                                      semG[b]).wait()

                @plsc.parallel_loop(0, _CHUNK, unroll=8)
                def _scale(r):
                    sb = plsc.load_gather(sv[b], [jnp.broadcast_to(r, (_L,))])
                    for q in range(8):
                        rows[b][r, pl.ds(q * _L, _L)] = (
                            rows[b][r, pl.ds(q * _L, _L)] * sb)

                pltpu.async_copy(rows[b], num_sh.at[scidx[b]], semS[b],
                                 add=True)

                @pl.when(c0 + b + 2 < nchunk)
                def _():
                    _start_idx(c0 + b + 2, b)
            return carry
        lax.fori_loop(0, nchunk // 2, _chunk_pair, 0)
        _wait_scatter(0)
        _wait_scatter(1)

        pltpu.sync_copy(den_v, den_out.at[pl.ds(pl.multiple_of(wid * n, 8), n)])
        plsc.subcore_barrier()

        obase = pl.multiple_of(cid * npad + base_sh, _CHUNK)
        for t in range(nfull):
            pltpu.sync_copy(num_sh.at[pl.ds(base_sh + t * _CHUNK, _CHUNK)],
                            num_out.at[pl.ds(obase + t * _CHUNK, _CHUNK)])

    return edge_kernel


def kernel(x, edge_index, edge_attr, W1, att_src1, att_dst1, W_e1, att_e1, b1,
           W2, att_src2, att_dst2, W_e2, att_e2, b2):
    n = x.shape[0]
    e = edge_index.shape[1]

    src = edge_index[0].astype(jnp.int32)
    dst = edge_index[1].astype(jnp.int32)
    ea = edge_attr[:, 0].astype(jnp.float32)

    ept = -(-e // (_NW * 2 * _CHUNK)) * (2 * _CHUNK)
    pad = _NW * ept - e
    src_p = jnp.pad(src, (0, pad))
    dst_p = jnp.pad(dst, (0, pad))
    ea_p = jnp.pad(ea, (0, pad))

    npad = -(-n // (_NSUB * _CHUNK)) * (_NSUB * _CHUNK)
    edge_kernel = _make_edge_kernel(n, e, ept)

    h1, s1, d1, ce1 = _run_dense1(
        x, W1, att_src1.reshape(128, 1), att_dst1.reshape(128, 1),
        W_e1, att_e1.reshape(1, 128), n)
    num1, den1 = edge_kernel(
        h1, s1.reshape(n), d1.reshape(n), src_p, dst_p, ea_p, ce1[0, :_L])

    h2, s2, d2, ce2 = _run_mid(
        num1.reshape(2, npad, 128), den1.reshape(_NW, n).T, b1.reshape(1, 128),
        W2, att_src2.reshape(128, 1), att_dst2.reshape(128, 1),
        W_e2, att_e2.reshape(1, 128), n)
    num2, den2 = edge_kernel(
        h2, s2.reshape(n), d2.reshape(n), src_p, dst_p, ea_p, ce2[0, :_L])

    out = _run_final(num2.reshape(2, npad, 128), den2.reshape(_NW, n).T,
                     b2.reshape(1, 128), n)
    return out


# speedup vs baseline: 32.1329x; 1.1641x over previous
import functools

import jax
import jax.numpy as jnp
from jax import lax
from jax.experimental import pallas as pl
from jax.experimental.pallas import tpu as pltpu
from jax.experimental.pallas import tpu_sc as plsc

_L = 16
_NSUB = 16
_NCORE = 2
_NW = _NSUB * _NCORE
_CHUNK = 64
_BLK = 2000


def _dense1_body(x_ref, w_ref, asr_ref, adr_ref, we_ref, ate_ref,
                 h_ref, s_ref, d_ref, ce_ref):
    h = jnp.dot(x_ref[...], w_ref[...], preferred_element_type=jnp.float32)
    h_ref[...] = h
    s_ref[...] = jnp.dot(h, asr_ref[...], preferred_element_type=jnp.float32)
    d_ref[...] = jnp.dot(h, adr_ref[...], preferred_element_type=jnp.float32)
    ce_ref[...] = jnp.full((1, 128), jnp.sum(we_ref[...] * ate_ref[...]),
                           dtype=jnp.float32)


def _mid_body(num_ref, den_ref, b_ref, w_ref, asr_ref, adr_ref, we_ref, ate_ref,
              h_ref, s_ref, d_ref, ce_ref):
    nm = num_ref[0] + num_ref[1]
    dn = jnp.sum(den_ref[...], axis=1, keepdims=True)
    h1 = nm / (dn + 1e-16) + b_ref[...]
    h1 = jnp.maximum(h1, 0.0)
    h = jnp.dot(h1, w_ref[...], preferred_element_type=jnp.float32)
    h_ref[...] = h
    s_ref[...] = jnp.dot(h, asr_ref[...], preferred_element_type=jnp.float32)
    d_ref[...] = jnp.dot(h, adr_ref[...], preferred_element_type=jnp.float32)
    ce_ref[...] = jnp.full((1, 128), jnp.sum(we_ref[...] * ate_ref[...]),
                           dtype=jnp.float32)


def _final_body(num_ref, den_ref, b_ref, out_ref):
    nm = num_ref[0] + num_ref[1]
    dn = jnp.sum(den_ref[...], axis=1, keepdims=True)
    out_ref[...] = nm / (dn + 1e-16) + b_ref[...]


def _run_dense1(x, W, att_s, att_d, We, att_e, n):
    grid = n // _BLK
    full = lambda i: (0, 0)
    row = lambda i: (i, 0)
    return pl.pallas_call(
        _dense1_body,
        grid=(grid,),
        in_specs=[
            pl.BlockSpec((_BLK, 128), row),
            pl.BlockSpec((128, 128), full),
            pl.BlockSpec((128, 1), full),
            pl.BlockSpec((128, 1), full),
            pl.BlockSpec((1, 128), full),
            pl.BlockSpec((1, 128), full),
        ],
        out_specs=[
            pl.BlockSpec((_BLK, 128), row),
            pl.BlockSpec((_BLK, 1), row),
            pl.BlockSpec((_BLK, 1), row),
            pl.BlockSpec((1, 128), full),
        ],
        out_shape=[
            jax.ShapeDtypeStruct((n, 128), jnp.float32),
            jax.ShapeDtypeStruct((n, 1), jnp.float32),
            jax.ShapeDtypeStruct((n, 1), jnp.float32),
            jax.ShapeDtypeStruct((1, 128), jnp.float32),
        ],
    )(x, W, att_s, att_d, We, att_e)


def _run_mid(num, den_t, b, W, att_s, att_d, We, att_e, n):
    grid = n // _BLK
    full = lambda i: (0, 0)
    row = lambda i: (i, 0)
    return pl.pallas_call(
        _mid_body,
        grid=(grid,),
        in_specs=[
            pl.BlockSpec((2, _BLK, 128), lambda i: (0, i, 0)),
            pl.BlockSpec((_BLK, 32), row),
            pl.BlockSpec((1, 128), full),
            pl.BlockSpec((128, 128), full),
            pl.BlockSpec((128, 1), full),
            pl.BlockSpec((128, 1), full),
            pl.BlockSpec((1, 128), full),
            pl.BlockSpec((1, 128), full),
        ],
        out_specs=[
            pl.BlockSpec((_BLK, 128), row),
            pl.BlockSpec((_BLK, 1), row),
            pl.BlockSpec((_BLK, 1), row),
            pl.BlockSpec((1, 128), full),
        ],
        out_shape=[
            jax.ShapeDtypeStruct((n, 128), jnp.float32),
            jax.ShapeDtypeStruct((n, 1), jnp.float32),
            jax.ShapeDtypeStruct((n, 1), jnp.float32),
            jax.ShapeDtypeStruct((1, 128), jnp.float32),
        ],
    )(num, den_t, b, W, att_s, att_d, We, att_e)


def _run_final(num, den_t, b, n):
    grid = n // _BLK
    full = lambda i: (0, 0)
    row = lambda i: (i, 0)
    return pl.pallas_call(
        _final_body,
        grid=(grid,),
        in_specs=[
            pl.BlockSpec((2, _BLK, 128), lambda i: (0, i, 0)),
            pl.BlockSpec((_BLK, 32), row),
            pl.BlockSpec((1, 128), full),
        ],
        out_specs=pl.BlockSpec((_BLK, 128), row),
        out_shape=jax.ShapeDtypeStruct((n, 128), jnp.float32),
    )(num, den_t, b)


def _make_edge_kernel(n, e, m0, m1):
    npad = -(-n // (_NSUB * _CHUNK)) * (_NSUB * _CHUNK)
    rows_per_tile = npad // _NSUB
    nfull = rows_per_tile // _CHUNK

    mesh = plsc.VectorSubcoreMesh(core_axis_name="c", subcore_axis_name="s")

    @functools.partial(
        pl.kernel,
        out_type=[
            jax.ShapeDtypeStruct((_NCORE * npad, 128), jnp.float32),
            jax.ShapeDtypeStruct((_NW * n,), jnp.float32),
        ],
        mesh=mesh,
        scratch_types=[
            pltpu.VMEM((n,), jnp.float32),
            pltpu.VMEM((n,), jnp.float32),
            pltpu.VMEM((n,), jnp.float32),
            [pltpu.VMEM((_CHUNK,), jnp.int32) for _ in range(2)],
            [pltpu.VMEM((_CHUNK,), jnp.int32) for _ in range(2)],
            [pltpu.VMEM((_CHUNK,), jnp.float32) for _ in range(2)],
            [pltpu.VMEM((_CHUNK,), jnp.int32) for _ in range(2)],
            [pltpu.VMEM((_CHUNK, 128), jnp.float32) for _ in range(2)],
            [pltpu.VMEM((_CHUNK,), jnp.float32) for _ in range(2)],
            pltpu.VMEM((_L,), jnp.float32),
            pltpu.VMEM_SHARED((npad, 128), jnp.float32),
            [pltpu.SemaphoreType.DMA for _ in range(6)],
        ],
        compiler_params=pltpu.CompilerParams(needs_layout_passes=False),
    )
    def edge_kernel(h_hbm, asrc_hbm, adst_hbm, src_hbm, dst_hbm, ea_hbm, ce_hbm,
                    num_out, den_out,
                    asrc_v, adst_v, den_v, srcv, dstv, eav, scidx, rows, sv,
                    cev, num_sh, sems):
        cid = lax.axis_index("c")
        sid = lax.axis_index("s")
        wid = cid * _NSUB + sid
        semI = sems[0:2]
        semG = sems[2:4]
        semS = sems[4:6]

        pltpu.sync_copy(asrc_hbm, asrc_v)
        pltpu.sync_copy(adst_hbm, adst_v)
        pltpu.sync_copy(ce_hbm, cev)
        cv = cev[...]

        zf = jnp.zeros((_L,), jnp.float32)

        def _zden(i, carry):
            den_v[pl.ds(i * _L, _L)] = zf
            return carry
        lax.fori_loop(0, n // _L, _zden, 0)

        def _zrows(r, carry):
            for q in range(8):
                rows[0][r, pl.ds(q * _L, _L)] = zf
            return carry
        lax.fori_loop(0, _CHUNK, _zrows, 0)

        base_sh = pl.multiple_of(sid * rows_per_tile, _CHUNK)
        for t in range(nfull):
            pltpu.sync_copy(rows[0],
                            num_sh.at[pl.ds(base_sh + t * _CHUNK, _CHUNK)])
        plsc.subcore_barrier()

        my_m = jnp.where(cid == 0, m0, m1)
        ebase = jnp.where(cid == 0, sid * m0,
                          _NSUB * m0 + sid * m1) * _CHUNK

        def _start_idx(c, b):
            base = ebase + c * _CHUNK
            pltpu.async_copy(src_hbm.at[pl.ds(base, _CHUNK)], srcv[b], semI[b])
            pltpu.async_copy(dst_hbm.at[pl.ds(base, _CHUNK)], dstv[b], semI[b])
            pltpu.async_copy(ea_hbm.at[pl.ds(base, _CHUNK)], eav[b], semI[b])

        def _wait_idx(c, b):
            base = ebase + c * _CHUNK
            pltpu.make_async_copy(src_hbm.at[pl.ds(base, _CHUNK)], srcv[b],
                                  semI[b]).wait()
            pltpu.make_async_copy(dst_hbm.at[pl.ds(base, _CHUNK)], dstv[b],
                                  semI[b]).wait()
            pltpu.make_async_copy(ea_hbm.at[pl.ds(base, _CHUNK)], eav[b],
                                  semI[b]).wait()

        def _wait_scatter(b):
            pltpu.make_async_copy(rows[b], num_sh.at[scidx[b]], semS[b]).wait()

        _start_idx(0, 0)
        _start_idx(1, 1)

        def _chunk_pair(t, carry):
            c0 = 2 * t
            for b in range(2):
                _wait_idx(c0 + b, b)
                @pl.when(t > 0)
                def _():
                    _wait_scatter(b)
                pltpu.async_copy(h_hbm.at[srcv[b]], rows[b], semG[b])

            for b in range(2):
                base = ebase + (c0 + b) * _CHUNK

                @plsc.parallel_loop(0, _CHUNK // _L, unroll=4)
                def _sbody(j):
                    si = srcv[b][pl.ds(j * _L, _L)]
                    di = dstv[b][pl.ds(j * _L, _L)]
                    av = plsc.load_gather(asrc_v, [si])
                    bv = plsc.load_gather(adst_v, [di])
                    al = av + bv + eav[b][pl.ds(j * _L, _L)] * cv
                    al = jnp.maximum(al, 0.2 * al)
                    sval = jnp.exp(al)
                    gid = base + j * _L + lax.iota(jnp.int32, _L)
                    sval = jnp.where(gid < e, sval, 0.0)
                    sv[b][pl.ds(j * _L, _L)] = sval
                    plsc.addupdate_scatter(den_v, [di], sval)

                for q in range(_CHUNK // _L):
                    scidx[b][pl.ds(q * _L, _L)] = dstv[b][pl.ds(q * _L, _L)]

            for b in range(2):
                pltpu.make_async_copy(h_hbm.at[srcv[b]], rows[b],
                                      semG[b]).wait()

                @plsc.parallel_loop(0, _CHUNK, unroll=8)
                def _scale(r):
                    sb = plsc.load_gather(sv[b], [jnp.broadcast_to(r, (_L,))])
                    for q in range(8):
                        rows[b][r, pl.ds(q * _L, _L)] = (
                            rows[b][r, pl.ds(q * _L, _L)] * sb)

                pltpu.async_copy(rows[b], num_sh.at[scidx[b]], semS[b],
                                 add=True)

                @pl.when(c0 + b + 2 < my_m)
                def _():
                    _start_idx(c0 + b + 2, b)
            return carry
        lax.fori_loop(0, my_m // 2, _chunk_pair, 0)
        _wait_scatter(0)
        _wait_scatter(1)

        pltpu.sync_copy(den_v, den_out.at[pl.ds(pl.multiple_of(wid * n, 8), n)])
        plsc.subcore_barrier()

        obase = pl.multiple_of(cid * npad + base_sh, _CHUNK)
        for t in range(nfull):
            pltpu.sync_copy(num_sh.at[pl.ds(base_sh + t * _CHUNK, _CHUNK)],
                            num_out.at[pl.ds(obase + t * _CHUNK, _CHUNK)])

    return edge_kernel


def kernel(x, edge_index, edge_attr, W1, att_src1, att_dst1, W_e1, att_e1, b1,
           W2, att_src2, att_dst2, W_e2, att_e2, b2):
    n = x.shape[0]
    e = edge_index.shape[1]

    src = edge_index[0].astype(jnp.int32)
    dst = edge_index[1].astype(jnp.int32)
    ea = edge_attr[:, 0].astype(jnp.float32)

    m_tot = 2 * (-(-e // (_NSUB * _CHUNK * 2)))
    m0 = max(2, 2 * round(0.363 * m_tot / 2))
    m1 = m_tot - m0
    pad = _NSUB * m_tot * _CHUNK - e
    src_p = jnp.pad(src, (0, pad))
    dst_p = jnp.pad(dst, (0, pad))
    ea_p = jnp.pad(ea, (0, pad))

    npad = -(-n // (_NSUB * _CHUNK)) * (_NSUB * _CHUNK)
    edge_kernel = _make_edge_kernel(n, e, m0, m1)

    h1, s1, d1, ce1 = _run_dense1(
        x, W1, att_src1.reshape(128, 1), att_dst1.reshape(128, 1),
        W_e1, att_e1.reshape(1, 128), n)
    num1, den1 = edge_kernel(
        h1, s1.reshape(n), d1.reshape(n), src_p, dst_p, ea_p, ce1[0, :_L])

    h2, s2, d2, ce2 = _run_mid(
        num1.reshape(2, npad, 128), den1.reshape(_NW, n).T, b1.reshape(1, 128),
        W2, att_src2.reshape(128, 1), att_dst2.reshape(128, 1),
        W_e2, att_e2.reshape(1, 128), n)
    num2, den2 = edge_kernel(
        h2, s2.reshape(n), d2.reshape(n), src_p, dst_p, ea_p, ce2[0, :_L])

    out = _run_final(num2.reshape(2, npad, 128), den2.reshape(_NW, n).T,
                     b2.reshape(1, 128), n)
    return out
